# Initial kernel scaffold; baseline (speedup 1.0000x reference)
import jax
import jax.numpy as jnp
from jax.experimental import pallas as pl


def kernel(neighbor_index, vertices, feature_map, weights, bias, directions):
    raise NotImplementedError("write your pallas kernel here")


# speedup vs baseline: 21.8779x; 21.8779x over previous
import functools

import jax
import jax.numpy as jnp
from jax import lax
from jax.experimental import pallas as pl
from jax.experimental.pallas import tpu as pltpu
from jax.experimental.pallas import tpu_sc as plsc

IN_CH = 128
OUT_CH = 64
BS = 2
V = 25000
NN = 16

CV = 40
ROWS = CV * NN
NCHUNK = (BS * V) // CV
NW = 32
ITERS = -(-NCHUNK // NW)
NSEG = ROWS // 128


def _tc_matmul(fm2d, weights, bias, vtx4):
    rows = fm2d.shape[0]
    blk = 1000
    grid = rows // blk

    def body(x_ref, w_ref, b_ref, v_ref, fc_ref, cb_ref):
        out = jnp.dot(x_ref[...], w_ref[...],
                      preferred_element_type=jnp.float32) + b_ref[...]
        fc_ref[...] = out[:, :OUT_CH]
        vpad = jnp.pad(v_ref[...], ((0, 0), (0, OUT_CH - 4)))
        cb_ref[...] = jnp.concatenate([out[:, OUT_CH:], vpad], axis=1)

    return pl.pallas_call(
        body,
        grid=(grid,),
        in_specs=[
            pl.BlockSpec((blk, IN_CH), lambda i: (i, 0)),
            pl.BlockSpec((IN_CH, 2 * OUT_CH), lambda i: (0, 0)),
            pl.BlockSpec((1, 2 * OUT_CH), lambda i: (0, 0)),
            pl.BlockSpec((blk, 4), lambda i: (i, 0)),
        ],
        out_specs=[
            pl.BlockSpec((blk, OUT_CH), lambda i: (i, 0)),
            pl.BlockSpec((blk, 2 * OUT_CH), lambda i: (i, 0)),
        ],
        out_shape=[
            jax.ShapeDtypeStruct((rows, OUT_CH), jnp.float32),
            jax.ShapeDtypeStruct((rows, 2 * OUT_CH), jnp.float32),
        ],
    )(fm2d, weights, bias.reshape(1, 2 * OUT_CH), vtx4)


def _rsqrt16(x):
    i = lax.bitcast_convert_type(x, jnp.int32)
    i = jnp.int32(0x5F3759DF) - (i >> 1)
    y = lax.bitcast_convert_type(i, jnp.float32)
    for _ in range(3):
        y = y * (1.5 - 0.5 * x * y * y)
    return y


def _make_sc_kernel():
    mesh = plsc.VectorSubcoreMesh(core_axis_name="c", subcore_axis_name="s")

    @functools.partial(
        pl.kernel,
        mesh=mesh,
        compiler_params=pltpu.CompilerParams(needs_layout_passes=False),
        out_type=jax.ShapeDtypeStruct((BS * V, OUT_CH), jnp.float32),
        scratch_types=[
            pltpu.VMEM((ROWS,), jnp.int32),
            pltpu.VMEM((ROWS, 2 * OUT_CH), jnp.float32),
            pltpu.VMEM((CV,), jnp.float32),
            pltpu.VMEM((CV,), jnp.float32),
            pltpu.VMEM((CV,), jnp.float32),
            pltpu.VMEM((CV, OUT_CH), jnp.float32),
            pltpu.VMEM((3 * OUT_CH,), jnp.float32),
            pltpu.VMEM((16,), jnp.float32),
            pltpu.VMEM((16,), jnp.float32),
            pltpu.VMEM((16,), jnp.float32),
            pltpu.SemaphoreType.DMA,
        ],
    )
    def sc_kernel(idx_hbm, xs_hbm, ys_hbm, zs_hbm, cb_hbm, fc_hbm, dirs_hbm,
                  out_hbm,
                  idx_v, cb_v, cx_v, cy_v, cz_v, fc_v,
                  dirs_v, dnx_s, dny_s, dnz_s, sem):
        wid = lax.axis_index("s") * 2 + lax.axis_index("c")
        li = lax.iota(jnp.int32, 16)

        pltpu.sync_copy(dirs_hbm, dirs_v)
        S = [[dirs_v[pl.ds(k * OUT_CH + oc * 16, 16)] for oc in range(4)]
             for k in range(3)]
        for oc in range(4):
            n2 = S[0][oc] * S[0][oc] + S[1][oc] * S[1][oc] + S[2][oc] * S[2][oc]
            rs = _rsqrt16(jnp.maximum(n2, 1e-24))
            for k in range(3):
                S[k][oc] = S[k][oc] * rs

        c0 = jnp.zeros((16,), jnp.int32)
        cxc = jnp.full((16,), OUT_CH, jnp.int32)
        cyc = jnp.full((16,), OUT_CH + 1, jnp.int32)
        czc = jnp.full((16,), OUT_CH + 2, jnp.int32)
        nsplat = [jnp.full((16,), n, jnp.int32) for n in range(NN)]
        neg_inf = jnp.full((16,), -jnp.inf, jnp.float32)

        def vbody(v, _):
            vs = c0 + v
            cx = plsc.load_gather(cx_v, [vs])
            cy = plsc.load_gather(cy_v, [vs])
            cz = plsc.load_gather(cz_v, [vs])
            ridx = li + v * NN
            nx = plsc.load_gather(cb_v, [ridx, cxc])
            ny = plsc.load_gather(cb_v, [ridx, cyc])
            nz = plsc.load_gather(cb_v, [ridx, czc])
            dx = nx - cx
            dy = ny - cy
            dz = nz - cz
            n2 = dx * dx + dy * dy + dz * dz
            rs = _rsqrt16(jnp.maximum(n2, 1e-24))
            dnx = dx * rs
            dny = dy * rs
            dnz = dz * rs
            acc = [neg_inf for _ in range(4)]
            for n in range(NN):
                a = dnx[n]
                b = dny[n]
                c = dnz[n]
                row = v * NN + n
                for oc in range(4):
                    f = cb_v[row, pl.ds(oc * 16, 16)]
                    t = a * S[0][oc] + b * S[1][oc] + c * S[2][oc]
                    t = jnp.maximum(t, 0.0)
                    acc[oc] = jnp.maximum(acc[oc], t * f)
            for oc in range(4):
                sl = pl.ds(oc * 16, 16)
                fc_v[v, sl] = fc_v[v, sl] + acc[oc]
            return 0

        def cbody(it, _):
            ci = wid + it * NW

            @pl.when(ci < NCHUNK)
            def _():
                r0 = ci * CV
                pltpu.sync_copy(idx_hbm.at[pl.ds(ci * ROWS, ROWS)], idx_v)
                pltpu.sync_copy(xs_hbm.at[pl.ds(r0, CV)], cx_v)
                pltpu.sync_copy(ys_hbm.at[pl.ds(r0, CV)], cy_v)
                pltpu.sync_copy(zs_hbm.at[pl.ds(r0, CV)], cz_v)
                pltpu.sync_copy(fc_hbm.at[pl.ds(r0, CV)], fc_v)
                cps = []
                for j in range(NSEG):
                    seg = pl.ds(j * 128, 128)
                    cps.append(pltpu.async_copy(
                        cb_hbm.at[idx_v.at[seg]], cb_v.at[seg], sem))
                for cp in cps:
                    cp.wait()
                lax.fori_loop(0, CV, vbody, 0)
                pltpu.sync_copy(fc_v, out_hbm.at[pl.ds(r0, CV)])

            return 0

        lax.fori_loop(0, ITERS, cbody, 0)

    return sc_kernel


_SC_KERNEL = _make_sc_kernel()


def kernel(neighbor_index, vertices, feature_map, weights, bias, directions):
    fm2d = feature_map.reshape(BS * V, IN_CH)
    vtx = vertices.reshape(BS * V, 3)
    vtx4 = jnp.pad(vtx, ((0, 0), (0, 1)))
    fc, cb = _tc_matmul(fm2d, weights, bias, vtx4)
    xs = vtx[:, 0]
    ys = vtx[:, 1]
    zs = vtx[:, 2]
    idx = (neighbor_index
           + (jnp.arange(BS, dtype=jnp.int32) * V).reshape(BS, 1, 1)
           ).reshape(NCHUNK * ROWS)
    out = _SC_KERNEL(idx, xs, ys, zs, cb, fc, directions.reshape(3 * OUT_CH))
    return out.reshape(BS, V, OUT_CH)


# speedup vs baseline: 24.2794x; 1.1098x over previous
import functools

import jax
import jax.numpy as jnp
from jax import lax
from jax.experimental import pallas as pl
from jax.experimental.pallas import tpu as pltpu
from jax.experimental.pallas import tpu_sc as plsc

IN_CH = 128
OUT_CH = 64
BS = 2
V = 25000
NN = 16

CV = 40
ROWS = CV * NN
NCHUNK = (BS * V) // CV
NW = 32
ITERS = -(-NCHUNK // NW)
NSEG = ROWS // 128


def _tc_matmul(fm2d, weights, bias, vtx):
    rows = fm2d.shape[0]
    blk = 1000
    grid = rows // blk

    def body(x_ref, w_ref, b_ref, v_ref, fc_ref, cb_ref):
        out = jnp.dot(x_ref[...], w_ref[...],
                      preferred_element_type=jnp.float32) + b_ref[...]
        fc_ref[...] = out[:, :OUT_CH]
        vpad = jnp.pad(v_ref[...], ((0, 0), (0, OUT_CH - 3)))
        cb_ref[...] = jnp.concatenate([out[:, OUT_CH:], vpad], axis=1)

    return pl.pallas_call(
        body,
        grid=(grid,),
        in_specs=[
            pl.BlockSpec((blk, IN_CH), lambda i: (i, 0)),
            pl.BlockSpec((IN_CH, 2 * OUT_CH), lambda i: (0, 0)),
            pl.BlockSpec((1, 2 * OUT_CH), lambda i: (0, 0)),
            pl.BlockSpec((blk, 3), lambda i: (i, 0)),
        ],
        out_specs=[
            pl.BlockSpec((blk, OUT_CH), lambda i: (i, 0)),
            pl.BlockSpec((blk, 2 * OUT_CH), lambda i: (i, 0)),
        ],
        out_shape=[
            jax.ShapeDtypeStruct((rows, OUT_CH), jnp.float32),
            jax.ShapeDtypeStruct((rows, 2 * OUT_CH), jnp.float32),
        ],
    )(fm2d, weights, bias.reshape(1, 2 * OUT_CH), vtx)


def _rsqrt16(x):
    i = lax.bitcast_convert_type(x, jnp.int32)
    i = jnp.int32(0x5F3759DF) - (i >> 1)
    y = lax.bitcast_convert_type(i, jnp.float32)
    for _ in range(3):
        y = y * (1.5 - 0.5 * x * y * y)
    return y


def _make_sc_kernel():
    mesh = plsc.VectorSubcoreMesh(core_axis_name="c", subcore_axis_name="s")

    @functools.partial(
        pl.kernel,
        mesh=mesh,
        compiler_params=pltpu.CompilerParams(needs_layout_passes=False),
        out_type=jax.ShapeDtypeStruct((BS * V, OUT_CH), jnp.float32),
        scratch_types=[
            pltpu.VMEM((ROWS,), jnp.int32),
            pltpu.VMEM((ROWS, 2 * OUT_CH), jnp.float32),
            pltpu.VMEM((CV, 2 * OUT_CH), jnp.float32),
            pltpu.VMEM((CV, OUT_CH), jnp.float32),
            pltpu.VMEM((3 * OUT_CH,), jnp.float32),
            pltpu.SemaphoreType.DMA,
        ],
    )
    def sc_kernel(idx_hbm, cb_hbm, fc_hbm, dirs_hbm, out_hbm,
                  idx_v, cb_v, ctr_v, fc_v, dirs_v, sem):
        wid = lax.axis_index("s") * 2 + lax.axis_index("c")
        li = lax.iota(jnp.int32, 16)

        pltpu.sync_copy(dirs_hbm, dirs_v)
        S = [[dirs_v[pl.ds(k * OUT_CH + oc * 16, 16)] for oc in range(4)]
             for k in range(3)]
        for oc in range(4):
            n2 = S[0][oc] * S[0][oc] + S[1][oc] * S[1][oc] + S[2][oc] * S[2][oc]
            rs = _rsqrt16(jnp.maximum(n2, 1e-24))
            for k in range(3):
                S[k][oc] = S[k][oc] * rs

        c0 = jnp.zeros((16,), jnp.int32)
        cxc = jnp.full((16,), OUT_CH, jnp.int32)
        cyc = jnp.full((16,), OUT_CH + 1, jnp.int32)
        czc = jnp.full((16,), OUT_CH + 2, jnp.int32)
        neg_inf = jnp.full((16,), -jnp.inf, jnp.float32)

        def vbody(v, _):
            vs = c0 + v
            cx = plsc.load_gather(ctr_v, [vs, cxc])
            cy = plsc.load_gather(ctr_v, [vs, cyc])
            cz = plsc.load_gather(ctr_v, [vs, czc])
            ridx = li + v * NN
            nx = plsc.load_gather(cb_v, [ridx, cxc])
            ny = plsc.load_gather(cb_v, [ridx, cyc])
            nz = plsc.load_gather(cb_v, [ridx, czc])
            dx = nx - cx
            dy = ny - cy
            dz = nz - cz
            n2 = dx * dx + dy * dy + dz * dz
            rs = _rsqrt16(jnp.maximum(n2, 1e-24))
            dnx = dx * rs
            dny = dy * rs
            dnz = dz * rs
            acc = [neg_inf for _ in range(4)]
            for n in range(NN):
                a = dnx[n]
                b = dny[n]
                c = dnz[n]
                row = v * NN + n
                for oc in range(4):
                    f = cb_v[row, pl.ds(oc * 16, 16)]
                    t = a * S[0][oc] + b * S[1][oc] + c * S[2][oc]
                    t = jnp.maximum(t, 0.0)
                    acc[oc] = jnp.maximum(acc[oc], t * f)
            for oc in range(4):
                sl = pl.ds(oc * 16, 16)
                fc_v[v, sl] = fc_v[v, sl] + acc[oc]
            return 0

        def cbody(it, _):
            ci = wid + it * NW

            @pl.when(ci < NCHUNK)
            def _():
                r0 = ci * CV
                pltpu.sync_copy(idx_hbm.at[pl.ds(ci * ROWS, ROWS)], idx_v)
                pltpu.sync_copy(cb_hbm.at[pl.ds(r0, CV)], ctr_v)
                pltpu.sync_copy(fc_hbm.at[pl.ds(r0, CV)], fc_v)
                cps = []
                for j in range(NSEG):
                    seg = pl.ds(j * 128, 128)
                    cps.append(pltpu.async_copy(
                        cb_hbm.at[idx_v.at[seg]], cb_v.at[seg], sem))
                for cp in cps:
                    cp.wait()
                lax.fori_loop(0, CV, vbody, 0)
                pltpu.sync_copy(fc_v, out_hbm.at[pl.ds(r0, CV)])

            return 0

        lax.fori_loop(0, ITERS, cbody, 0)

    return sc_kernel


_SC_KERNEL = _make_sc_kernel()


def kernel(neighbor_index, vertices, feature_map, weights, bias, directions):
    fm2d = feature_map.reshape(BS * V, IN_CH)
    vtx = vertices.reshape(BS * V, 3)
    fc, cb = _tc_matmul(fm2d, weights, bias, vtx)
    idx = (neighbor_index
           + (jnp.arange(BS, dtype=jnp.int32) * V).reshape(BS, 1, 1)
           ).reshape(NCHUNK * ROWS)
    out = _SC_KERNEL(idx, cb, fc, directions.reshape(3 * OUT_CH))
    return out.reshape(BS, V, OUT_CH)


# speedup vs baseline: 25.4984x; 1.0502x over previous
import functools

import jax
import jax.numpy as jnp
from jax import lax
from jax.experimental import pallas as pl
from jax.experimental.pallas import tpu as pltpu
from jax.experimental.pallas import tpu_sc as plsc

IN_CH = 128
OUT_CH = 64
BS = 2
V = 25000
NN = 16

CV = 16
ROWS = CV * NN
NCHUNK = (BS * V) // CV
NW = 32
ITERS = -(-NCHUNK // NW)
PAIRS = -(-ITERS // 2)
SEGS = [(o, min(128, ROWS - o)) for o in range(0, ROWS, 128)]


def _tc_matmul(fm2d, weights, bias, vtx):
    rows = fm2d.shape[0]
    blk = 1000
    grid = rows // blk

    def body(x_ref, w_ref, b_ref, v_ref, fc_ref, cb_ref):
        out = jnp.dot(x_ref[...], w_ref[...],
                      preferred_element_type=jnp.float32) + b_ref[...]
        fc_ref[...] = out[:, :OUT_CH]
        vpad = jnp.pad(v_ref[...], ((0, 0), (0, OUT_CH - 3)))
        cb_ref[...] = jnp.concatenate([out[:, OUT_CH:], vpad], axis=1)

    return pl.pallas_call(
        body,
        grid=(grid,),
        in_specs=[
            pl.BlockSpec((blk, IN_CH), lambda i: (i, 0)),
            pl.BlockSpec((IN_CH, 2 * OUT_CH), lambda i: (0, 0)),
            pl.BlockSpec((1, 2 * OUT_CH), lambda i: (0, 0)),
            pl.BlockSpec((blk, 3), lambda i: (i, 0)),
        ],
        out_specs=[
            pl.BlockSpec((blk, OUT_CH), lambda i: (i, 0)),
            pl.BlockSpec((blk, 2 * OUT_CH), lambda i: (i, 0)),
        ],
        out_shape=[
            jax.ShapeDtypeStruct((rows, OUT_CH), jnp.float32),
            jax.ShapeDtypeStruct((rows, 2 * OUT_CH), jnp.float32),
        ],
    )(fm2d, weights, bias.reshape(1, 2 * OUT_CH), vtx)


def _rsqrt16(x):
    i = lax.bitcast_convert_type(x, jnp.int32)
    i = jnp.int32(0x5F3759DF) - (i >> 1)
    y = lax.bitcast_convert_type(i, jnp.float32)
    for _ in range(3):
        y = y * (1.5 - 0.5 * x * y * y)
    return y


def _make_sc_kernel():
    mesh = plsc.VectorSubcoreMesh(core_axis_name="c", subcore_axis_name="s")

    @functools.partial(
        pl.kernel,
        mesh=mesh,
        compiler_params=pltpu.CompilerParams(needs_layout_passes=False),
        out_type=jax.ShapeDtypeStruct((BS * V, OUT_CH), jnp.float32),
        scratch_types=[
            pltpu.VMEM((ROWS,), jnp.int32),
            pltpu.VMEM((ROWS,), jnp.int32),
            pltpu.VMEM((ROWS, 2 * OUT_CH), jnp.float32),
            pltpu.VMEM((ROWS, 2 * OUT_CH), jnp.float32),
            pltpu.VMEM((CV, 2 * OUT_CH), jnp.float32),
            pltpu.VMEM((CV, OUT_CH), jnp.float32),
            pltpu.VMEM((3 * OUT_CH,), jnp.float32),
            pltpu.SemaphoreType.DMA,
            pltpu.SemaphoreType.DMA,
        ],
    )
    def sc_kernel(idx_hbm, cb_hbm, fc_hbm, dirs_hbm, out_hbm,
                  idx_a, idx_b, cb_a, cb_b, ctr_v, fc_v, dirs_v,
                  sem_a, sem_b):
        wid = lax.axis_index("s") * 2 + lax.axis_index("c")
        li = lax.iota(jnp.int32, 16)

        pltpu.sync_copy(dirs_hbm, dirs_v)
        S = [[dirs_v[pl.ds(k * OUT_CH + oc * 16, 16)] for oc in range(4)]
             for k in range(3)]
        for oc in range(4):
            n2 = S[0][oc] * S[0][oc] + S[1][oc] * S[1][oc] + S[2][oc] * S[2][oc]
            rs = _rsqrt16(jnp.maximum(n2, 1e-24))
            for k in range(3):
                S[k][oc] = S[k][oc] * rs

        c0 = jnp.zeros((16,), jnp.int32)
        cxc = jnp.full((16,), OUT_CH, jnp.int32)
        cyc = jnp.full((16,), OUT_CH + 1, jnp.int32)
        czc = jnp.full((16,), OUT_CH + 2, jnp.int32)
        neg_inf = jnp.full((16,), -jnp.inf, jnp.float32)

        def issue(ci, idx_v, cb_v, sem):
            pltpu.sync_copy(idx_hbm.at[pl.ds(ci * ROWS, ROWS)], idx_v)
            for off, sz in SEGS:
                seg = pl.ds(off, sz)
                pltpu.async_copy(cb_hbm.at[idx_v.at[seg]], cb_v.at[seg], sem)

        def drain(idx_v, cb_v, sem):
            for off, sz in SEGS:
                seg = pl.ds(off, sz)
                pltpu.make_async_copy(
                    cb_hbm.at[idx_v.at[seg]], cb_v.at[seg], sem).wait()

        def compute(ci, cb_v):
            r0 = ci * CV
            pltpu.sync_copy(cb_hbm.at[pl.ds(r0, CV)], ctr_v)
            pltpu.sync_copy(fc_hbm.at[pl.ds(r0, CV)], fc_v)

            def vbody(v, _):
                vs = c0 + v
                cx = plsc.load_gather(ctr_v, [vs, cxc])
                cy = plsc.load_gather(ctr_v, [vs, cyc])
                cz = plsc.load_gather(ctr_v, [vs, czc])
                ridx = li + v * NN
                nx = plsc.load_gather(cb_v, [ridx, cxc])
                ny = plsc.load_gather(cb_v, [ridx, cyc])
                nz = plsc.load_gather(cb_v, [ridx, czc])
                dx = nx - cx
                dy = ny - cy
                dz = nz - cz
                n2 = dx * dx + dy * dy + dz * dz
                rs = _rsqrt16(jnp.maximum(n2, 1e-24))
                dnx = dx * rs
                dny = dy * rs
                dnz = dz * rs
                acc = [neg_inf for _ in range(4)]
                for n in range(NN):
                    a = dnx[n]
                    b = dny[n]
                    c = dnz[n]
                    row = v * NN + n
                    for oc in range(4):
                        f = cb_v[row, pl.ds(oc * 16, 16)]
                        t = a * S[0][oc] + b * S[1][oc] + c * S[2][oc]
                        t = jnp.maximum(t, 0.0)
                        acc[oc] = jnp.maximum(acc[oc], t * f)
                for oc in range(4):
                    sl = pl.ds(oc * 16, 16)
                    fc_v[v, sl] = fc_v[v, sl] + acc[oc]
                return 0

            lax.fori_loop(0, CV, vbody, 0)
            pltpu.sync_copy(fc_v, out_hbm.at[pl.ds(r0, CV)])

        issue(wid, idx_a, cb_a, sem_a)

        def pbody(j, _):
            ci0 = wid + (2 * j) * NW
            ci1 = ci0 + NW
            ci2 = ci0 + 2 * NW

            @pl.when(ci0 < NCHUNK)
            def _():
                drain(idx_a, cb_a, sem_a)

                @pl.when(ci1 < NCHUNK)
                def _():
                    issue(ci1, idx_b, cb_b, sem_b)

                compute(ci0, cb_a)

            @pl.when(ci1 < NCHUNK)
            def _():
                drain(idx_b, cb_b, sem_b)

                @pl.when(ci2 < NCHUNK)
                def _():
                    issue(ci2, idx_a, cb_a, sem_a)

                compute(ci1, cb_b)

            return 0

        lax.fori_loop(0, PAIRS, pbody, 0)

    return sc_kernel


_SC_KERNEL = _make_sc_kernel()


def kernel(neighbor_index, vertices, feature_map, weights, bias, directions):
    fm2d = feature_map.reshape(BS * V, IN_CH)
    vtx = vertices.reshape(BS * V, 3)
    fc, cb = _tc_matmul(fm2d, weights, bias, vtx)
    idx = (neighbor_index
           + (jnp.arange(BS, dtype=jnp.int32) * V).reshape(BS, 1, 1)
           ).reshape(NCHUNK * ROWS)
    out = _SC_KERNEL(idx, cb, fc, directions.reshape(3 * OUT_CH))
    return out.reshape(BS, V, OUT_CH)


# speedup vs baseline: 34.4228x; 1.3500x over previous
import functools

import jax
import jax.numpy as jnp
from jax import lax
from jax.experimental import pallas as pl
from jax.experimental.pallas import tpu as pltpu
from jax.experimental.pallas import tpu_sc as plsc

IN_CH = 128
OUT_CH = 64
BS = 2
V = 25000
NN = 16

CV = 16
ROWS = CV * NN
NCHUNK = (BS * V) // CV
NW = 32
ITERS = -(-NCHUNK // NW)
PAIRS = -(-ITERS // 2)
SEGS = [(o, min(128, ROWS - o)) for o in range(0, ROWS, 128)]


def _tc_matmul(fm2d, weights, bias, vtx):
    rows = fm2d.shape[0]
    blk = 1000
    grid = rows // blk

    def body(x_ref, w_ref, b_ref, v_ref, fc_ref, cb_ref):
        out = jnp.dot(x_ref[...], w_ref[...],
                      preferred_element_type=jnp.float32) + b_ref[...]
        fc_ref[...] = out[:, :OUT_CH]
        vpad = jnp.pad(v_ref[...], ((0, 0), (0, OUT_CH - 3)))
        cb_ref[...] = jnp.concatenate([out[:, OUT_CH:], vpad], axis=1)

    return pl.pallas_call(
        body,
        grid=(grid,),
        in_specs=[
            pl.BlockSpec((blk, IN_CH), lambda i: (i, 0)),
            pl.BlockSpec((IN_CH, 2 * OUT_CH), lambda i: (0, 0)),
            pl.BlockSpec((1, 2 * OUT_CH), lambda i: (0, 0)),
            pl.BlockSpec((blk, 3), lambda i: (i, 0)),
        ],
        out_specs=[
            pl.BlockSpec((blk, OUT_CH), lambda i: (i, 0)),
            pl.BlockSpec((blk, 2 * OUT_CH), lambda i: (i, 0)),
        ],
        out_shape=[
            jax.ShapeDtypeStruct((rows, OUT_CH), jnp.float32),
            jax.ShapeDtypeStruct((rows, 2 * OUT_CH), jnp.float32),
        ],
    )(fm2d, weights, bias.reshape(1, 2 * OUT_CH), vtx)


def _rsqrt16(x):
    i = lax.bitcast_convert_type(x, jnp.int32)
    i = jnp.int32(0x5F3759DF) - (i >> 1)
    y = lax.bitcast_convert_type(i, jnp.float32)
    for _ in range(2):
        y = y * (1.5 - 0.5 * x * y * y)
    return y


def _make_sc_kernel():
    mesh = plsc.VectorSubcoreMesh(core_axis_name="c", subcore_axis_name="s")

    @functools.partial(
        pl.kernel,
        mesh=mesh,
        compiler_params=pltpu.CompilerParams(needs_layout_passes=False),
        out_type=jax.ShapeDtypeStruct((BS * V, OUT_CH), jnp.float32),
        scratch_types=[
            pltpu.VMEM((ROWS,), jnp.int32),
            pltpu.VMEM((ROWS,), jnp.int32),
            pltpu.VMEM((ROWS, 2 * OUT_CH), jnp.float32),
            pltpu.VMEM((ROWS, 2 * OUT_CH), jnp.float32),
            pltpu.VMEM((CV, 2 * OUT_CH), jnp.float32),
            pltpu.VMEM((CV, 2 * OUT_CH), jnp.float32),
            pltpu.VMEM((CV, OUT_CH), jnp.float32),
            pltpu.VMEM((CV, OUT_CH), jnp.float32),
            pltpu.VMEM((3 * OUT_CH,), jnp.float32),
            pltpu.SemaphoreType.DMA,
            pltpu.SemaphoreType.DMA,
        ],
    )
    def sc_kernel(idx_hbm, cb_hbm, fc_hbm, dirs_hbm, out_hbm,
                  idx_a, idx_b, cb_a, cb_b, ctr_a, ctr_b, fc_a, fc_b,
                  dirs_v, sem_a, sem_b):
        wid = lax.axis_index("s") * 2 + lax.axis_index("c")
        li = lax.iota(jnp.int32, 16)

        pltpu.sync_copy(dirs_hbm, dirs_v)
        S = [[dirs_v[pl.ds(k * OUT_CH + oc * 16, 16)] for oc in range(4)]
             for k in range(3)]
        for oc in range(4):
            n2 = S[0][oc] * S[0][oc] + S[1][oc] * S[1][oc] + S[2][oc] * S[2][oc]
            rs = _rsqrt16(jnp.maximum(n2, 1e-24))
            for k in range(3):
                S[k][oc] = S[k][oc] * rs

        c0 = jnp.zeros((16,), jnp.int32)
        cxc = jnp.full((16,), OUT_CH, jnp.int32)
        cyc = jnp.full((16,), OUT_CH + 1, jnp.int32)
        czc = jnp.full((16,), OUT_CH + 2, jnp.int32)
        neg_inf = jnp.full((16,), -jnp.inf, jnp.float32)

        def issue(ci, idx_v, cb_v, ctr_v, fc_v, sem):
            r0 = ci * CV
            pltpu.sync_copy(idx_hbm.at[pl.ds(ci * ROWS, ROWS)], idx_v)
            for off, sz in SEGS:
                seg = pl.ds(off, sz)
                pltpu.async_copy(cb_hbm.at[idx_v.at[seg]], cb_v.at[seg], sem)
            pltpu.async_copy(cb_hbm.at[pl.ds(r0, CV)], ctr_v, sem)
            pltpu.async_copy(fc_hbm.at[pl.ds(r0, CV)], fc_v, sem)

        def drain(ci, idx_v, cb_v, ctr_v, fc_v, sem):
            r0 = ci * CV
            for off, sz in SEGS:
                seg = pl.ds(off, sz)
                pltpu.make_async_copy(
                    cb_hbm.at[idx_v.at[seg]], cb_v.at[seg], sem).wait()
            pltpu.make_async_copy(
                cb_hbm.at[pl.ds(r0, CV)], ctr_v, sem).wait()
            pltpu.make_async_copy(
                fc_hbm.at[pl.ds(r0, CV)], fc_v, sem).wait()

        def compute(ci, cb_v, ctr_v, fc_v):
            r0 = ci * CV

            def vbody(v, _):
                vs = c0 + v
                cx = plsc.load_gather(ctr_v, [vs, cxc])
                cy = plsc.load_gather(ctr_v, [vs, cyc])
                cz = plsc.load_gather(ctr_v, [vs, czc])
                ridx = li + v * NN
                nx = plsc.load_gather(cb_v, [ridx, cxc])
                ny = plsc.load_gather(cb_v, [ridx, cyc])
                nz = plsc.load_gather(cb_v, [ridx, czc])
                dx = nx - cx
                dy = ny - cy
                dz = nz - cz
                n2 = dx * dx + dy * dy + dz * dz
                rs = _rsqrt16(jnp.maximum(n2, 1e-24))
                dnx = dx * rs
                dny = dy * rs
                dnz = dz * rs
                acc = [neg_inf for _ in range(4)]
                for n in range(NN):
                    a = dnx[n]
                    b = dny[n]
                    c = dnz[n]
                    row = v * NN + n
                    for oc in range(4):
                        f = cb_v[row, pl.ds(oc * 16, 16)]
                        t = a * S[0][oc] + b * S[1][oc] + c * S[2][oc]
                        t = jnp.maximum(t, 0.0)
                        acc[oc] = jnp.maximum(acc[oc], t * f)
                for oc in range(4):
                    sl = pl.ds(oc * 16, 16)
                    fc_v[v, sl] = fc_v[v, sl] + acc[oc]
                return 0

            lax.fori_loop(0, CV, vbody, 0)
            pltpu.sync_copy(fc_v, out_hbm.at[pl.ds(r0, CV)])

        issue(wid, idx_a, cb_a, ctr_a, fc_a, sem_a)

        def pbody(j, _):
            ci0 = wid + (2 * j) * NW
            ci1 = ci0 + NW
            ci2 = ci0 + 2 * NW

            @pl.when(ci0 < NCHUNK)
            def _():
                drain(ci0, idx_a, cb_a, ctr_a, fc_a, sem_a)

                @pl.when(ci1 < NCHUNK)
                def _():
                    issue(ci1, idx_b, cb_b, ctr_b, fc_b, sem_b)

                compute(ci0, cb_a, ctr_a, fc_a)

            @pl.when(ci1 < NCHUNK)
            def _():
                drain(ci1, idx_b, cb_b, ctr_b, fc_b, sem_b)

                @pl.when(ci2 < NCHUNK)
                def _():
                    issue(ci2, idx_a, cb_a, ctr_a, fc_a, sem_a)

                compute(ci1, cb_b, ctr_b, fc_b)

            return 0

        lax.fori_loop(0, PAIRS, pbody, 0)

    return sc_kernel


_SC_KERNEL = _make_sc_kernel()


def kernel(neighbor_index, vertices, feature_map, weights, bias, directions):
    fm2d = feature_map.reshape(BS * V, IN_CH)
    vtx = vertices.reshape(BS * V, 3)
    fc, cb = _tc_matmul(fm2d, weights, bias, vtx)
    idx = (neighbor_index
           + (jnp.arange(BS, dtype=jnp.int32) * V).reshape(BS, 1, 1)
           ).reshape(NCHUNK * ROWS)
    out = _SC_KERNEL(idx, cb, fc, directions.reshape(3 * OUT_CH))
    return out.reshape(BS, V, OUT_CH)
